# Initial kernel scaffold; baseline (speedup 1.0000x reference)
import jax
import jax.numpy as jnp
from jax.experimental import pallas as pl


def kernel(inputs, embed_table, W, b):
    raise NotImplementedError("write your pallas kernel here")


# speedup vs baseline: 1.9957x; 1.9957x over previous
import functools

import jax
import jax.numpy as jnp
from jax import lax
from jax.experimental import pallas as pl
from jax.experimental.pallas import tpu as pltpu
from jax.experimental.pallas import tpu_sc as plsc

DIM = 32
ROWS_PER_BLOCK = 25000

NUM_CORES = 2
NUM_SUBCORES = 16
NUM_WORKERS = NUM_CORES * NUM_SUBCORES
CHUNK = 128


def _scores_body(tbl_ref, w_ref, b_ref, out_ref):
    x = tbl_ref[:]
    w = w_ref[:]
    s = jnp.sum(x * w, axis=1, keepdims=True) + b_ref[0, 0]
    out_ref[:] = jax.nn.sigmoid(s)


def _compute_scores(embed_table, W, b):
    vocab = embed_table.shape[0]
    grid = vocab // ROWS_PER_BLOCK
    return pl.pallas_call(
        _scores_body,
        grid=(grid,),
        in_specs=[
            pl.BlockSpec((ROWS_PER_BLOCK, DIM), lambda i: (i, 0)),
            pl.BlockSpec((1, DIM), lambda i: (0, 0)),
            pl.BlockSpec(memory_space=pltpu.SMEM),
        ],
        out_specs=pl.BlockSpec((ROWS_PER_BLOCK, 1), lambda i: (i, 0)),
        out_shape=jax.ShapeDtypeStruct((vocab, 1), jnp.float32),
    )(embed_table, W, b.reshape(1, 1))


def _make_gather(n_total):
    per_worker = n_total // NUM_WORKERS
    n_chunks = per_worker // CHUNK
    rows_total = n_total // CHUNK
    mesh = plsc.VectorSubcoreMesh(core_axis_name="c", subcore_axis_name="s")

    @functools.partial(
        pl.kernel,
        mesh=mesh,
        out_type=jax.ShapeDtypeStruct((rows_total, CHUNK), jnp.float32),
        scratch_types=[
            pltpu.VMEM((n_chunks, CHUNK), jnp.int32),
            pltpu.VMEM((n_chunks, CHUNK), jnp.float32),
            pltpu.SemaphoreType.DMA,
        ],
    )
    def gather_kernel(scores_hbm, idx_hbm, out_hbm, idx_v, out_v, sem):
        wid = lax.axis_index("s") * NUM_CORES + lax.axis_index("c")
        rbase = wid * n_chunks
        pltpu.sync_copy(idx_hbm.at[pl.ds(rbase, n_chunks)], idx_v)

        def _start(j, carry):
            pltpu.make_async_copy(
                scores_hbm.at[idx_v.at[j]], out_v.at[j], sem
            ).start()
            return carry

        lax.fori_loop(0, n_chunks, _start, 0)

        def _drain(j, carry):
            pltpu.make_async_copy(
                scores_hbm.at[idx_v.at[j]], out_v.at[j], sem
            ).wait()
            return carry

        lax.fori_loop(0, n_chunks, _drain, 0)
        pltpu.sync_copy(out_v, out_hbm.at[pl.ds(rbase, n_chunks)])

    return gather_kernel


def kernel(inputs, embed_table, W, b):
    bsz, seq = inputs.shape
    n_total = bsz * seq
    scores = _compute_scores(embed_table, W, b)
    idx2d = inputs.astype(jnp.int32).reshape(n_total // CHUNK, CHUNK)
    out2d = _make_gather(n_total)(scores.reshape(-1), idx2d)
    return out2d.reshape(bsz, seq, 1)


# speedup vs baseline: 10.0660x; 5.0439x over previous
import functools

import jax
import jax.numpy as jnp
from jax import lax
from jax.experimental import pallas as pl
from jax.experimental.pallas import tpu as pltpu
from jax.experimental.pallas import tpu_sc as plsc

DIM = 32
COLS_PER_BLOCK = 8192

NUM_CORES = 2
NUM_SUBCORES = 16
NUM_WORKERS = NUM_CORES * NUM_SUBCORES
CHUNK = 128


def _scores_body(tbl_ref, w_ref, b_ref, out_ref):
    x = tbl_ref[:]
    w = w_ref[:]
    s = jnp.sum(x * w, axis=0) + b_ref[0]
    out_ref[:] = jax.nn.sigmoid(s)


def _compute_scores(table_t, W, b):
    vocab = table_t.shape[1]
    grid = (vocab + COLS_PER_BLOCK - 1) // COLS_PER_BLOCK
    return pl.pallas_call(
        _scores_body,
        grid=(grid,),
        in_specs=[
            pl.BlockSpec((DIM, COLS_PER_BLOCK), lambda i: (0, i)),
            pl.BlockSpec((DIM, 1), lambda i: (0, 0)),
            pl.BlockSpec(memory_space=pltpu.SMEM),
        ],
        out_specs=pl.BlockSpec((COLS_PER_BLOCK,), lambda i: (i,)),
        out_shape=jax.ShapeDtypeStruct((vocab,), jnp.float32),
    )(table_t, W.reshape(DIM, 1), b)


def _make_gather(n_total):
    per_worker = n_total // NUM_WORKERS
    n_chunks = per_worker // CHUNK
    rows_total = n_total // CHUNK
    mesh = plsc.VectorSubcoreMesh(core_axis_name="c", subcore_axis_name="s")

    @functools.partial(
        pl.kernel,
        mesh=mesh,
        out_type=jax.ShapeDtypeStruct((rows_total, CHUNK), jnp.float32),
        scratch_types=[
            pltpu.VMEM((n_chunks, CHUNK), jnp.int32),
            pltpu.VMEM((n_chunks, CHUNK), jnp.float32),
            pltpu.SemaphoreType.DMA,
        ],
    )
    def gather_kernel(scores_hbm, idx_hbm, out_hbm, idx_v, out_v, sem):
        wid = lax.axis_index("s") * NUM_CORES + lax.axis_index("c")
        rbase = wid * n_chunks
        pltpu.sync_copy(idx_hbm.at[pl.ds(rbase, n_chunks)], idx_v)

        def _start(j, carry):
            pltpu.make_async_copy(
                scores_hbm.at[idx_v.at[j]], out_v.at[j], sem
            ).start()
            return carry

        lax.fori_loop(0, n_chunks, _start, 0)

        def _drain(j, carry):
            pltpu.make_async_copy(
                scores_hbm.at[idx_v.at[j]], out_v.at[j], sem
            ).wait()
            return carry

        lax.fori_loop(0, n_chunks, _drain, 0)
        pltpu.sync_copy(out_v, out_hbm.at[pl.ds(rbase, n_chunks)])

    return gather_kernel


def kernel(inputs, embed_table, W, b):
    bsz, seq = inputs.shape
    n_total = bsz * seq
    scores = _compute_scores(embed_table.T, W, b)
    idx2d = inputs.T.astype(jnp.int32).reshape(n_total // CHUNK, CHUNK)
    out2d = _make_gather(n_total)(scores, idx2d)
    return jnp.transpose(out2d.reshape(seq, bsz))[:, :, None]


# speedup vs baseline: 12.0980x; 1.2019x over previous
import functools

import jax
import jax.numpy as jnp
from jax import lax
from jax.experimental import pallas as pl
from jax.experimental.pallas import tpu as pltpu
from jax.experimental.pallas import tpu_sc as plsc

DIM = 32
COLS_PER_BLOCK = 16384

NUM_CORES = 2
NUM_SUBCORES = 16
NUM_WORKERS = NUM_CORES * NUM_SUBCORES
CHUNK = 128


def _scores_body(tbl_ref, w_ref, b_ref, out_ref):
    x = tbl_ref[:]
    w = w_ref[:]
    s = jnp.sum(x * w, axis=0) + b_ref[0]
    out_ref[:] = jax.nn.sigmoid(s)


def _compute_scores(table_t, W, b):
    vocab = table_t.shape[1]
    grid = (vocab + COLS_PER_BLOCK - 1) // COLS_PER_BLOCK
    return pl.pallas_call(
        _scores_body,
        grid=(grid,),
        in_specs=[
            pl.BlockSpec((DIM, COLS_PER_BLOCK), lambda i: (0, i)),
            pl.BlockSpec((DIM, 1), lambda i: (0, 0)),
            pl.BlockSpec(memory_space=pltpu.SMEM),
        ],
        out_specs=pl.BlockSpec((COLS_PER_BLOCK,), lambda i: (i,)),
        out_shape=jax.ShapeDtypeStruct((vocab,), jnp.float32),
    )(table_t, W.reshape(DIM, 1), b)


def _make_gather(n_total):
    per_worker = n_total // NUM_WORKERS
    n_chunks = per_worker // CHUNK
    mesh = plsc.VectorSubcoreMesh(core_axis_name="c", subcore_axis_name="s")

    @functools.partial(
        pl.kernel,
        mesh=mesh,
        out_type=jax.ShapeDtypeStruct((n_total,), jnp.float32),
        scratch_types=[
            pltpu.VMEM((per_worker,), jnp.int32),
            pltpu.VMEM((per_worker,), jnp.float32),
            pltpu.SemaphoreType.DMA,
        ],
    )
    def gather_kernel(scores_hbm, idx_hbm, out_hbm, idx_v, out_v, sem):
        wid = lax.axis_index("s") * NUM_CORES + lax.axis_index("c")
        base = wid * per_worker
        pltpu.sync_copy(idx_hbm.at[pl.ds(base, per_worker)], idx_v)

        def _start(j, carry):
            pltpu.make_async_copy(
                scores_hbm.at[idx_v.at[pl.ds(j * CHUNK, CHUNK)]],
                out_v.at[pl.ds(j * CHUNK, CHUNK)],
                sem,
            ).start()
            return carry

        lax.fori_loop(0, n_chunks, _start, 0)

        def _drain(j, carry):
            pltpu.make_async_copy(
                scores_hbm.at[idx_v.at[pl.ds(j * CHUNK, CHUNK)]],
                out_v.at[pl.ds(j * CHUNK, CHUNK)],
                sem,
            ).wait()
            return carry

        lax.fori_loop(0, n_chunks, _drain, 0)
        pltpu.sync_copy(out_v, out_hbm.at[pl.ds(base, per_worker)])

    return gather_kernel


def kernel(inputs, embed_table, W, b):
    bsz, seq = inputs.shape
    n_total = bsz * seq
    scores = _compute_scores(embed_table.T, W, b)
    idx_flat = inputs.T.astype(jnp.int32).reshape(n_total)
    out_flat = _make_gather(n_total)(scores, idx_flat)
    return jnp.transpose(out_flat.reshape(seq, bsz))[:, :, None]


# speedup vs baseline: 13.7533x; 1.1368x over previous
import functools

import jax
import jax.numpy as jnp
from jax import lax
from jax.experimental import pallas as pl
from jax.experimental.pallas import tpu as pltpu
from jax.experimental.pallas import tpu_sc as plsc

DIM = 32
COLS_PER_BLOCK = 32768

NUM_CORES = 2
NUM_SUBCORES = 16
NUM_WORKERS = NUM_CORES * NUM_SUBCORES
CHUNK = 128


def _scores_body(tbl_ref, w_ref, b_ref, out_ref):
    x = tbl_ref[:]
    w = w_ref[:]
    s = jnp.sum(x * w, axis=0) + b_ref[0]
    out_ref[:] = jax.nn.sigmoid(s)


def _compute_scores(table_t, W, b):
    vocab = table_t.shape[1]
    grid = (vocab + COLS_PER_BLOCK - 1) // COLS_PER_BLOCK
    return pl.pallas_call(
        _scores_body,
        grid=(grid,),
        in_specs=[
            pl.BlockSpec((DIM, COLS_PER_BLOCK), lambda i: (0, i)),
            pl.BlockSpec((DIM, 1), lambda i: (0, 0)),
            pl.BlockSpec(memory_space=pltpu.SMEM),
        ],
        out_specs=pl.BlockSpec((COLS_PER_BLOCK,), lambda i: (i,)),
        out_shape=jax.ShapeDtypeStruct((vocab,), jnp.float32),
    )(table_t, W.reshape(DIM, 1), b)


def _make_gather(n_total):
    per_worker = n_total // NUM_WORKERS
    n_chunks = per_worker // CHUNK
    mesh = plsc.VectorSubcoreMesh(core_axis_name="c", subcore_axis_name="s")

    @functools.partial(
        pl.kernel,
        mesh=mesh,
        out_type=jax.ShapeDtypeStruct((n_total,), jnp.float32),
        scratch_types=[
            pltpu.VMEM((per_worker,), jnp.int32),
            pltpu.VMEM((per_worker,), jnp.float32),
            pltpu.SemaphoreType.DMA,
        ],
    )
    def gather_kernel(scores_hbm, idx_hbm, out_hbm, idx_v, out_v, sem):
        wid = lax.axis_index("s") * NUM_CORES + lax.axis_index("c")
        base = wid * per_worker
        pltpu.sync_copy(idx_hbm.at[pl.ds(base, per_worker)], idx_v)
        pltpu.async_copy(scores_hbm.at[idx_v], out_v, sem).wait()
        pltpu.sync_copy(out_v, out_hbm.at[pl.ds(base, per_worker)])

    return gather_kernel


def kernel(inputs, embed_table, W, b):
    bsz, seq = inputs.shape
    n_total = bsz * seq
    scores = _compute_scores(embed_table.T, W, b)
    idx_flat = inputs.T.astype(jnp.int32).reshape(n_total)
    out_flat = _make_gather(n_total)(scores, idx_flat)
    return jnp.transpose(out_flat.reshape(seq, bsz))[:, :, None]


# speedup vs baseline: 14.5573x; 1.0585x over previous
import functools

import jax
import jax.numpy as jnp
from jax import lax
from jax.experimental import pallas as pl
from jax.experimental.pallas import tpu as pltpu
from jax.experimental.pallas import tpu_sc as plsc

DIM = 32
COLS_PER_BLOCK = 65536

NUM_CORES = 2
NUM_SUBCORES = 16
NUM_WORKERS = NUM_CORES * NUM_SUBCORES
CHUNK = 128


def _scores_body(tbl_ref, w_ref, b_ref, out_ref):
    x = tbl_ref[:]
    w = w_ref[:]
    s = jnp.sum(x * w, axis=0) + b_ref[0]
    out_ref[:] = jax.nn.sigmoid(s)


def _compute_scores(table_t, W, b):
    vocab = table_t.shape[1]
    grid = (vocab + COLS_PER_BLOCK - 1) // COLS_PER_BLOCK
    return pl.pallas_call(
        _scores_body,
        grid=(grid,),
        in_specs=[
            pl.BlockSpec((DIM, COLS_PER_BLOCK), lambda i: (0, i)),
            pl.BlockSpec((DIM, 1), lambda i: (0, 0)),
            pl.BlockSpec(memory_space=pltpu.SMEM),
        ],
        out_specs=pl.BlockSpec((COLS_PER_BLOCK,), lambda i: (i,)),
        out_shape=jax.ShapeDtypeStruct((vocab,), jnp.float32),
    )(table_t, W.reshape(DIM, 1), b)


def _make_gather(n_total):
    per_worker = n_total // NUM_WORKERS
    n_chunks = per_worker // CHUNK
    mesh = plsc.VectorSubcoreMesh(core_axis_name="c", subcore_axis_name="s")

    @functools.partial(
        pl.kernel,
        mesh=mesh,
        out_type=jax.ShapeDtypeStruct((n_total,), jnp.float32),
        scratch_types=[
            pltpu.VMEM((per_worker,), jnp.int32),
            pltpu.VMEM((per_worker,), jnp.float32),
            pltpu.SemaphoreType.DMA,
        ],
    )
    def gather_kernel(scores_hbm, idx_hbm, out_hbm, idx_v, out_v, sem):
        wid = lax.axis_index("s") * NUM_CORES + lax.axis_index("c")
        base = wid * per_worker
        pltpu.sync_copy(idx_hbm.at[pl.ds(base, per_worker)], idx_v)
        pltpu.async_copy(scores_hbm.at[idx_v], out_v, sem).wait()
        pltpu.sync_copy(out_v, out_hbm.at[pl.ds(base, per_worker)])

    return gather_kernel


def kernel(inputs, embed_table, W, b):
    bsz, seq = inputs.shape
    n_total = bsz * seq
    scores = _compute_scores(embed_table.T, W, b)
    idx_flat = inputs.T.astype(jnp.int32).reshape(n_total)
    out_flat = _make_gather(n_total)(scores, idx_flat)
    return jnp.transpose(out_flat.reshape(seq, bsz))[:, :, None]


# speedup vs baseline: 16.7240x; 1.1488x over previous
import functools

import jax
import jax.numpy as jnp
from jax import lax
from jax.experimental import pallas as pl
from jax.experimental.pallas import tpu as pltpu
from jax.experimental.pallas import tpu_sc as plsc

DIM = 32
COLS_PER_BLOCK = 131072

NUM_CORES = 2
NUM_SUBCORES = 16
NUM_WORKERS = NUM_CORES * NUM_SUBCORES
CHUNK = 128


def _scores_body(tbl_ref, w_ref, b_ref, out_ref):
    x = tbl_ref[:]
    w = w_ref[:]
    s = jnp.sum(x * w, axis=0) + b_ref[0]
    out_ref[:] = jax.nn.sigmoid(s)


def _compute_scores(table_t, W, b):
    vocab = table_t.shape[1]
    grid = (vocab + COLS_PER_BLOCK - 1) // COLS_PER_BLOCK
    return pl.pallas_call(
        _scores_body,
        grid=(grid,),
        in_specs=[
            pl.BlockSpec((DIM, COLS_PER_BLOCK), lambda i: (0, i)),
            pl.BlockSpec((DIM, 1), lambda i: (0, 0)),
            pl.BlockSpec(memory_space=pltpu.SMEM),
        ],
        out_specs=pl.BlockSpec((COLS_PER_BLOCK,), lambda i: (i,)),
        out_shape=jax.ShapeDtypeStruct((vocab,), jnp.float32),
    )(table_t, W.reshape(DIM, 1), b)


def _make_gather(n_total):
    per_worker = n_total // NUM_WORKERS
    n_chunks = per_worker // CHUNK
    mesh = plsc.VectorSubcoreMesh(core_axis_name="c", subcore_axis_name="s")

    @functools.partial(
        pl.kernel,
        mesh=mesh,
        out_type=jax.ShapeDtypeStruct((n_total,), jnp.float32),
        scratch_types=[
            pltpu.VMEM((per_worker,), jnp.int32),
            pltpu.VMEM((per_worker,), jnp.float32),
            pltpu.SemaphoreType.DMA,
        ],
    )
    def gather_kernel(scores_hbm, idx_hbm, out_hbm, idx_v, out_v, sem):
        wid = lax.axis_index("s") * NUM_CORES + lax.axis_index("c")
        base = wid * per_worker
        pltpu.sync_copy(idx_hbm.at[pl.ds(base, per_worker)], idx_v)
        pltpu.async_copy(scores_hbm.at[idx_v], out_v, sem).wait()
        pltpu.sync_copy(out_v, out_hbm.at[pl.ds(base, per_worker)])

    return gather_kernel


def kernel(inputs, embed_table, W, b):
    bsz, seq = inputs.shape
    n_total = bsz * seq
    scores = _compute_scores(embed_table.T, W, b)
    idx_flat = inputs.T.astype(jnp.int32).reshape(n_total)
    out_flat = _make_gather(n_total)(scores, idx_flat)
    return jnp.transpose(out_flat.reshape(seq, 1, bsz), (2, 0, 1))
